# baseline (speedup 1.0000x reference)
import functools

import jax
import jax.numpy as jnp
from jax import lax
from jax.experimental import pallas as pl
from jax.experimental.pallas import tpu as pltpu
from jax.experimental.pallas import tpu_sc as plsc

NUM_EMB = 1000000
DIM = 32
B_TOK = 16384
SEQ = 50

NC = 2
NS = 16
NW = NC * NS
TOK_PER_W = B_TOK // NW
TB = 128
NTB = TOK_PER_W // TB
N_UNITS = SEQ * NTB
N_PAIRS = N_UNITS // 2


def _emb_body(idx_hbm, table_hbm, out_hbm,
              idx_v, idxT_v, rows_v0, rows_v1, rows_v2, rows_v3,
              tile_v0, tile_v1, gsem, ssem):
    wid = lax.axis_index("s") * NC + lax.axis_index("c")
    tok0 = wid * TOK_PER_W
    iota = lax.iota(jnp.int32, 16)

    pltpu.sync_copy(idx_hbm.at[pl.ds(tok0, TOK_PER_W)], idx_v)

    def tr_idx(s, carry):
        col = jnp.full((16,), s, jnp.int32)
        for c in range(TOK_PER_W // 16):
            v = plsc.load_gather(idx_v, [c * 16 + iota, col])
            idxT_v[pl.ds(s * TOK_PER_W + c * 16, 16)] = v
        return carry

    lax.fori_loop(0, SEQ, tr_idx, 0, unroll=False)


    def fire_gather(g, rows_v):
        pltpu.async_copy(table_hbm.at[idxT_v.at[pl.ds(g * TB, TB)]], rows_v, gsem)

    def drain_gather(g, rows_v):
        pltpu.make_async_copy(
            table_hbm.at[idxT_v.at[pl.ds(g * TB, TB)]], rows_v, gsem
        ).wait()

    def out_slices(g):
        s = g // NTB
        th = wid * NTB + lax.rem(g, NTB)
        return s, th

    def fire_stores(g, tile_v):
        s, th = out_slices(g)
        for dh in range(DIM // 8):
            pltpu.async_copy(
                tile_v.at[pl.ds(dh * 8, 8)], out_hbm.at[s, dh, th], ssem
            )

    def drain_stores(g, tile_v):
        s, th = out_slices(g)
        for dh in range(DIM // 8):
            pltpu.make_async_copy(
                tile_v.at[pl.ds(dh * 8, 8)], out_hbm.at[s, dh, th], ssem
            ).wait()

    cols_all = [jnp.full((16,), d, jnp.int32) for d in range(DIM)]

    def transpose(rows_v, tile_v):
        for c in range(TB // 16):
            sub = rows_v.at[pl.ds(c * 16, 16)]
            for d0 in range(0, DIM, 4):
                vs = [plsc.load_gather(sub, [iota, cols_all[d0 + k]])
                      for k in range(4)]
                for k in range(4):
                    tile_v[d0 + k, pl.ds(c * 16, 16)] = vs[k]

    rows_bufs = [rows_v0, rows_v1, rows_v2, rows_v3]
    tile_bufs = [tile_v0, tile_v1]

    def unit(g, j, first, last):
        if not last:
            fire_gather(g + 2, rows_bufs[(j + 2) % 4])
        drain_gather(g, rows_bufs[j])
        if not first:
            drain_stores(g - 2, tile_bufs[j % 2])
        transpose(rows_bufs[j], tile_bufs[j % 2])
        fire_stores(g, tile_bufs[j % 2])

    fire_gather(0, rows_v0)
    fire_gather(1, rows_v1)
    unit(0, 0, first=True, last=False)
    unit(1, 1, first=True, last=False)

    def quad(q, carry):
        g0 = 2 + 4 * q
        for j in range(4):
            unit(g0 + j, (2 + j) % 4, first=False, last=False)
        return carry

    lax.fori_loop(0, (N_UNITS - 4) // 4, quad, 0, unroll=False)

    unit(N_UNITS - 2, (N_UNITS - 2) % 4, first=False, last=True)
    unit(N_UNITS - 1, (N_UNITS - 1) % 4, first=False, last=True)
    drain_stores(N_UNITS - 2, tile_bufs[(N_UNITS - 2) % 2])
    drain_stores(N_UNITS - 1, tile_bufs[(N_UNITS - 1) % 2])


@jax.jit
def _emb(idx, weight):
    mesh = plsc.VectorSubcoreMesh(core_axis_name="c", subcore_axis_name="s")
    run = pl.kernel(
        _emb_body,
        out_type=jax.ShapeDtypeStruct((SEQ, DIM // 8, B_TOK // TB, 8, TB),
                                      jnp.float32),
        mesh=mesh,
        scratch_types=[
            pltpu.VMEM((TOK_PER_W, SEQ), jnp.int32),
            pltpu.VMEM((TOK_PER_W * SEQ,), jnp.int32),
            pltpu.VMEM((TB, DIM), jnp.float32),
            pltpu.VMEM((TB, DIM), jnp.float32),
            pltpu.VMEM((TB, DIM), jnp.float32),
            pltpu.VMEM((TB, DIM), jnp.float32),
            pltpu.VMEM((DIM, TB), jnp.float32),
            pltpu.VMEM((DIM, TB), jnp.float32),
            pltpu.SemaphoreType.DMA,
            pltpu.SemaphoreType.DMA,
        ],
        compiler_params=pltpu.CompilerParams(
            use_tc_tiling_on_sc=False, needs_layout_passes=False
        ),
    )
    phys = run(idx, weight)
    return phys.transpose(2, 4, 0, 1, 3).reshape(B_TOK, SEQ, DIM)


def kernel(token_ids, weight):
    return _emb(token_ids.astype(jnp.int32), weight)


# speedup vs baseline: 1.0406x; 1.0406x over previous
import functools

import jax
import jax.numpy as jnp
from jax import lax
from jax.experimental import pallas as pl
from jax.experimental.pallas import tpu as pltpu
from jax.experimental.pallas import tpu_sc as plsc

NUM_EMB = 1000000
DIM = 32
B_TOK = 16384
SEQ = 50

NC = 2
NS = 16
NW = NC * NS
TOK_PER_W = B_TOK // NW
TB = 128
NTB = TOK_PER_W // TB
N_UNITS = SEQ * NTB
N_PAIRS = N_UNITS // 2


def _emb_body(idx_hbm, table_hbm, out_hbm,
              idx_v, idxT_v, rows_v0, rows_v1, rows_v2, rows_v3,
              tile_v0, tile_v1, gsem, ssem):
    wid = lax.axis_index("s") * NC + lax.axis_index("c")
    tok0 = wid * TOK_PER_W
    iota = lax.iota(jnp.int32, 16)

    pltpu.sync_copy(idx_hbm.at[pl.ds(tok0, TOK_PER_W)], idx_v)

    def tr_idx(s, carry):
        col = jnp.full((16,), s, jnp.int32)
        for c in range(TOK_PER_W // 16):
            v = plsc.load_gather(idx_v, [c * 16 + iota, col])
            idxT_v[pl.ds(s * TOK_PER_W + c * 16, 16)] = v
        return carry

    lax.fori_loop(0, SEQ, tr_idx, 0, unroll=False)


    def fire_gather(g, rows_v):
        pltpu.async_copy(table_hbm.at[idxT_v.at[pl.ds(g * TB, TB)]], rows_v, gsem)

    def drain_gather(g, rows_v):
        pltpu.make_async_copy(
            table_hbm.at[idxT_v.at[pl.ds(g * TB, TB)]], rows_v, gsem
        ).wait()

    def out_slices(g):
        s = g // NTB
        th = wid * NTB + lax.rem(g, NTB)
        return s, th

    def fire_stores(g, tile_v):
        s, th = out_slices(g)
        for dh in range(DIM // 8):
            pltpu.async_copy(
                tile_v.at[pl.ds(dh * 8, 8)], out_hbm.at[s, dh, th], ssem
            )

    def drain_stores(g, tile_v):
        s, th = out_slices(g)
        for dh in range(DIM // 8):
            pltpu.make_async_copy(
                tile_v.at[pl.ds(dh * 8, 8)], out_hbm.at[s, dh, th], ssem
            ).wait()

    cols_all = [jnp.full((16,), d, jnp.int32) for d in range(DIM)]

    def transpose(rows_v, tile_v):
        for c in range(TB // 16):
            sub = rows_v.at[pl.ds(c * 16, 16)]
            for d0 in range(0, DIM, 8):
                vs = [plsc.load_gather(sub, [iota, cols_all[d0 + k]])
                      for k in range(8)]
                for k in range(8):
                    tile_v[d0 + k, pl.ds(c * 16, 16)] = vs[k]

    rows_bufs = [rows_v0, rows_v1, rows_v2, rows_v3]
    tile_bufs = [tile_v0, tile_v1]

    def unit(g, j, first, last):
        if not last:
            fire_gather(g + 2, rows_bufs[(j + 2) % 4])
        drain_gather(g, rows_bufs[j])
        if not first:
            drain_stores(g - 2, tile_bufs[j % 2])
        transpose(rows_bufs[j], tile_bufs[j % 2])
        fire_stores(g, tile_bufs[j % 2])

    fire_gather(0, rows_v0)
    fire_gather(1, rows_v1)
    unit(0, 0, first=True, last=False)
    unit(1, 1, first=True, last=False)

    def quad(q, carry):
        g0 = 2 + 4 * q
        for j in range(4):
            unit(g0 + j, (2 + j) % 4, first=False, last=False)
        return carry

    lax.fori_loop(0, (N_UNITS - 4) // 4, quad, 0, unroll=False)

    unit(N_UNITS - 2, (N_UNITS - 2) % 4, first=False, last=True)
    unit(N_UNITS - 1, (N_UNITS - 1) % 4, first=False, last=True)
    drain_stores(N_UNITS - 2, tile_bufs[(N_UNITS - 2) % 2])
    drain_stores(N_UNITS - 1, tile_bufs[(N_UNITS - 1) % 2])


@jax.jit
def _emb(idx, weight):
    mesh = plsc.VectorSubcoreMesh(core_axis_name="c", subcore_axis_name="s")
    run = pl.kernel(
        _emb_body,
        out_type=jax.ShapeDtypeStruct((SEQ, DIM // 8, B_TOK // TB, 8, TB),
                                      jnp.float32),
        mesh=mesh,
        scratch_types=[
            pltpu.VMEM((TOK_PER_W, SEQ), jnp.int32),
            pltpu.VMEM((TOK_PER_W * SEQ,), jnp.int32),
            pltpu.VMEM((TB, DIM), jnp.float32),
            pltpu.VMEM((TB, DIM), jnp.float32),
            pltpu.VMEM((TB, DIM), jnp.float32),
            pltpu.VMEM((TB, DIM), jnp.float32),
            pltpu.VMEM((DIM, TB), jnp.float32),
            pltpu.VMEM((DIM, TB), jnp.float32),
            pltpu.SemaphoreType.DMA,
            pltpu.SemaphoreType.DMA,
        ],
        compiler_params=pltpu.CompilerParams(
            use_tc_tiling_on_sc=False, needs_layout_passes=False
        ),
    )
    phys = run(idx, weight)
    return phys.transpose(2, 4, 0, 1, 3).reshape(B_TOK, SEQ, DIM)


def kernel(token_ids, weight):
    return _emb(token_ids.astype(jnp.int32), weight)
